# baseline (speedup 1.0000x reference)
import jax
import jax.numpy as jnp
from jax import lax
from jax.experimental import pallas as pl
from jax.experimental.pallas import tpu as pltpu
from jax.experimental.pallas import tpu_sc as plsc

N = 10000
E = 320000
D = 128
EPS = 1e-5

NB = 10
BR = N // NB
NW = 32
ER = 5120
RPW = ER // NW
NPAD = 10016
RPS = NPAD // 16

_mesh = plsc.VectorSubcoreMesh(core_axis_name="c", subcore_axis_name="s")


def _sc_scatter(with_deg):
    out_type = [jax.ShapeDtypeStruct((NW, RPS, D), jnp.float32)]
    scratch = [
        pltpu.VMEM_SHARED((NPAD, D), jnp.float32),
        pltpu.VMEM((RPW, D), jnp.int32),
        pltpu.VMEM((D,), jnp.int32),
        pltpu.VMEM((D,), jnp.int32),
        pltpu.VMEM((D,), jnp.int32),
        pltpu.VMEM((D,), jnp.int32),
        pltpu.VMEM((D, D), jnp.float32),
        pltpu.SemaphoreType.DMA,
    ]
    if with_deg:
        out_type.append(jax.ShapeDtypeStruct((NW, RPS, 16), jnp.float32))
        scratch += [
            pltpu.VMEM_SHARED((NPAD, 16), jnp.float32),
            pltpu.VMEM((D, 16), jnp.float32),
        ]

    def body(vw1, packed, zeros, zeros16, ones_in, part, *rest):
        if with_deg:
            (degpart, acc_sh, packed_b, sx0, sx1, dx0, dx1, rows, gsem,
             deg_sh, ones_v) = rest
        else:
            acc_sh, packed_b, sx0, sx1, dx0, dx1, rows, gsem = rest
        srcx = [sx0, sx1]
        dstx = [dx0, dx1]
        cid = lax.axis_index("c")
        sid = lax.axis_index("s")
        w = cid * 16 + sid
        pltpu.sync_copy(packed.at[pl.ds(w * RPW, RPW)], packed_b)
        pltpu.sync_copy(zeros, acc_sh.at[pl.ds(sid * RPS, RPS)])
        if with_deg:
            pltpu.sync_copy(zeros16, deg_sh.at[pl.ds(sid * RPS, RPS)])
            pltpu.sync_copy(ones_in, ones_v)
        plsc.subcore_barrier()

        def unpack(t, q):
            for k in range(8):
                v = packed_b[t, pl.ds(k * 16, 16)]
                srcx[q][pl.ds(k * 16, 16)] = jnp.bitwise_and(v, 0xFFFF)
                dstx[q][pl.ds(k * 16, 16)] = lax.shift_right_logical(v, 16)

        def slot(t, q, do_unpack):
            h = pltpu.async_copy(vw1.at[srcx[q]], rows, gsem)
            if do_unpack:
                unpack(t + 1, 1 - q)
            h.wait()
            pltpu.sync_copy(rows, acc_sh.at[dstx[q]], add=True)
            if with_deg:
                pltpu.sync_copy(ones_v, deg_sh.at[dstx[q]], add=True)

        unpack(0, 0)

        def two_slots(c, carry):
            slot(2 * c, 0, True)
            slot(2 * c + 1, 1, True)
            return carry

        lax.fori_loop(0, RPW // 2 - 1, two_slots, 0)
        slot(RPW - 2, 0, True)
        slot(RPW - 1, 1, False)

        plsc.subcore_barrier()
        pltpu.sync_copy(acc_sh.at[pl.ds(sid * RPS, RPS)], part.at[w])
        if with_deg:
            pltpu.sync_copy(deg_sh.at[pl.ds(sid * RPS, RPS)], degpart.at[w])

    return pl.kernel(body, out_type=out_type, mesh=_mesh, scratch_types=scratch,
                     compiler_params=pltpu.CompilerParams(use_tc_tiling_on_sc=False))


_sc_scatter_deg = _sc_scatter(True)
_sc_scatter_nodeg = _sc_scatter(False)


def _mm2_body(x_ref, w0_ref, b0_ref, w1_ref, b1_ref, o0_ref, o1_ref):
    x = x_ref[...]
    o0_ref[...] = jnp.dot(x, w0_ref[...], preferred_element_type=jnp.float32) + b0_ref[...]
    o1_ref[...] = jnp.dot(x, w1_ref[...], preferred_element_type=jnp.float32) + b1_ref[...]


def _mm2(x, w0, b0, w1, b1):
    blk = pl.BlockSpec((BR, D), lambda i: (i, 0))
    wspec = pl.BlockSpec((D, D), lambda i: (0, 0))
    bspec = pl.BlockSpec((1, D), lambda i: (0, 0))
    return pl.pallas_call(
        _mm2_body,
        grid=(NB,),
        in_specs=[blk, wspec, bspec, wspec, bspec],
        out_specs=[blk, blk],
        out_shape=[jax.ShapeDtypeStruct((N, D), jnp.float32)] * 2,
    )(x, w0, b0.reshape(1, D), w1, b1.reshape(1, D))


def _fuse_factory(last):
    def body(vw0_ref, p_ref, degp_ref, g_ref, be_ref, res_ref,
             w0_ref, b0_ref, w1_ref, b1_ref, *orefs):
        if last:
            o0_ref, t_s, sums_s = orefs
            o1_ref = None
        else:
            o0_ref, o1_ref, t_s, sums_s = orefs
        p = pl.program_id(0)
        i = pl.program_id(1)

        @pl.when(p == 0)
        def _():
            deg = degp_ref[0, :, 0] + degp_ref[1, :, 0]
            dinv = 1.0 / (1.0 + deg)
            t = (vw0_ref[...] + p_ref[0] + p_ref[1]) * dinv[:, None]
            t_s[pl.ds(i * BR, BR), :] = t
            s = jnp.sum(t, axis=0)
            s2 = jnp.sum(t * t, axis=0)
            upd = jnp.concatenate(
                [s[None, :], s2[None, :], jnp.zeros((6, D), jnp.float32)], axis=0)

            @pl.when(i == 0)
            def _():
                sums_s[...] = upd

            @pl.when(i > 0)
            def _():
                sums_s[...] = sums_s[...] + upd

        @pl.when(p == 1)
        def _():
            m = sums_s[0, :] / N
            v = sums_s[1, :] / N - m * m
            scale = g_ref[0, :] * lax.rsqrt(v + EPS)
            t = t_s[pl.ds(i * BR, BR), :]
            y = (t - m[None, :]) * scale[None, :] + be_ref[0, :][None, :]
            if last:
                y = y + res_ref[...]
            y = jnp.maximum(y, 0.0)
            if last:
                o0_ref[...] = y
            else:
                o0_ref[...] = jnp.dot(
                    y, w0_ref[...], preferred_element_type=jnp.float32) + b0_ref[...]
                o1_ref[...] = jnp.dot(
                    y, w1_ref[...], preferred_element_type=jnp.float32) + b1_ref[...]

    blk = pl.BlockSpec((BR, D), lambda p, i: (i, 0))
    row = pl.BlockSpec((1, D), lambda p, i: (0, 0))
    wsp = pl.BlockSpec((D, D), lambda p, i: (0, 0))
    in_specs = [
        blk,
        pl.BlockSpec((2, BR, D), lambda p, i: (0, i, 0)),
        pl.BlockSpec((2, BR, 16), lambda p, i: (0, i, 0)),
        row, row,
        blk,
        wsp, row, wsp, row,
    ]
    n_out = 1 if last else 2
    return pl.pallas_call(
        body,
        grid=(2, NB),
        in_specs=in_specs,
        out_specs=[blk] * n_out,
        out_shape=[jax.ShapeDtypeStruct((N, D), jnp.float32)] * n_out,
        scratch_shapes=[pltpu.VMEM((N, D), jnp.float32),
                        pltpu.VMEM((8, D), jnp.float32)],
    )


_fuse_mid = _fuse_factory(False)
_fuse_last = _fuse_factory(True)


def kernel(features, edges, w0_0, b0_0, w1_0, b1_0, g_0, be_0,
           w0_1, b0_1, w1_1, b1_1, g_1, be_1,
           w0_2, b0_2, w1_2, b1_2, g_2, be_2):
    npad = ER * D - 2 * E
    srcs = jnp.concatenate(
        [edges[:, 1], edges[:, 0], jnp.zeros((npad,), jnp.int32)])
    dsts = jnp.concatenate(
        [edges[:, 0], edges[:, 1], jnp.full((npad,), N, jnp.int32)])
    packed = (srcs + dsts * 65536).reshape(ER, D)
    zeros = jnp.zeros((RPS, D), jnp.float32)
    zeros16 = jnp.zeros((RPS, 16), jnp.float32)
    ones16 = jnp.ones((D, 16), jnp.float32)

    params = [(w0_1, b0_1, w1_1, b1_1, g_0, be_0),
              (w0_2, b0_2, w1_2, b1_2, g_1, be_1),
              (w0_2, b0_2, w1_2, b1_2, g_2, be_2)]

    vw0, vw1 = _mm2(features, w0_0, b0_0, w1_0, b1_0)
    degpart = None
    for li, (w0n, b0n, w1n, b1n, g, be) in enumerate(params):
        if li == 0:
            part, degpart = _sc_scatter_deg(vw1, packed, zeros, zeros16, ones16)
            degpart = degpart.reshape(2, NPAD, 16)
        else:
            (part,) = _sc_scatter_nodeg(vw1, packed, zeros, zeros16, ones16)
        fuse = _fuse_last if li == 2 else _fuse_mid
        outs = fuse(vw0, part.reshape(2, NPAD, D), degpart,
                    g.reshape(1, D), be.reshape(1, D), features,
                    w0n, b0n.reshape(1, D), w1n, b1n.reshape(1, D))
        if li == 2:
            (y,) = outs
            return y
        vw0, vw1 = outs


# speedup vs baseline: 1.1705x; 1.1705x over previous
import jax
import jax.numpy as jnp
from jax import lax
from jax.experimental import pallas as pl
from jax.experimental.pallas import tpu as pltpu
from jax.experimental.pallas import tpu_sc as plsc

N = 10000
E = 320000
D = 128
EPS = 1e-5

NB = 10
BR = N // NB
NW = 32
ER = 5120
RPW = ER // NW
NPAD = 10016
RPS = NPAD // 16

_mesh = plsc.VectorSubcoreMesh(core_axis_name="c", subcore_axis_name="s")


def _sc_scatter(with_deg):
    out_type = [jax.ShapeDtypeStruct((NW, RPS, D), jnp.float32)]
    scratch = [
        pltpu.VMEM_SHARED((NPAD, D), jnp.float32),
        pltpu.VMEM((D,), jnp.int32),
        pltpu.VMEM((D,), jnp.int32),
        pltpu.VMEM((D,), jnp.int32),
        pltpu.VMEM((D,), jnp.int32),
        pltpu.VMEM((D, D), jnp.float32),
        pltpu.SemaphoreType.DMA,
        pltpu.SemaphoreType.DMA,
    ]
    if with_deg:
        out_type.append(jax.ShapeDtypeStruct((NW, RPS, 16), jnp.float32))
        scratch += [
            pltpu.VMEM_SHARED((NPAD, 16), jnp.float32),
            pltpu.VMEM((D, 16), jnp.float32),
        ]

    def body(vw1, srcs, dsts, zeros, zeros16, ones_in, part, *rest):
        if with_deg:
            (degpart, acc_sh, sx0, sx1, dx0, dx1, rows, gsem, isem,
             deg_sh, ones_v) = rest
        else:
            acc_sh, sx0, sx1, dx0, dx1, rows, gsem, isem = rest
        srcx = [sx0, sx1]
        dstx = [dx0, dx1]
        cid = lax.axis_index("c")
        sid = lax.axis_index("s")
        w = cid * 16 + sid
        base = w * RPW
        pltpu.sync_copy(zeros, acc_sh.at[pl.ds(sid * RPS, RPS)])
        if with_deg:
            pltpu.sync_copy(zeros16, deg_sh.at[pl.ds(sid * RPS, RPS)])
            pltpu.sync_copy(ones_in, ones_v)
        plsc.subcore_barrier()

        def idx_start(t, q):
            pltpu.async_copy(srcs.at[base + t], srcx[q], isem)
            pltpu.async_copy(dsts.at[base + t], dstx[q], isem)

        def idx_wait(t, q):
            pltpu.make_async_copy(srcs.at[base + t], srcx[q], isem).wait()
            pltpu.make_async_copy(dsts.at[base + t], dstx[q], isem).wait()

        def slot(t, q, first=False, last=False):
            if not first:
                idx_wait(t, q)
            h = pltpu.async_copy(vw1.at[srcx[q]], rows, gsem)
            if not last:
                idx_start(t + 1, 1 - q)
            h.wait()
            pltpu.sync_copy(rows, acc_sh.at[dstx[q]], add=True)
            if with_deg:
                pltpu.sync_copy(ones_v, deg_sh.at[dstx[q]], add=True)

        pltpu.sync_copy(srcs.at[base], sx0)
        pltpu.sync_copy(dsts.at[base], dx0)
        slot(0, 0, first=True)

        def two_slots(c, carry):
            slot(2 * c + 1, 1)
            slot(2 * c + 2, 0)
            return carry

        lax.fori_loop(0, RPW // 2 - 1, two_slots, 0)
        slot(RPW - 1, 1, last=True)

        plsc.subcore_barrier()
        pltpu.sync_copy(acc_sh.at[pl.ds(sid * RPS, RPS)], part.at[w])
        if with_deg:
            pltpu.sync_copy(deg_sh.at[pl.ds(sid * RPS, RPS)], degpart.at[w])

    return pl.kernel(body, out_type=out_type, mesh=_mesh, scratch_types=scratch,
                     compiler_params=pltpu.CompilerParams(use_tc_tiling_on_sc=False))


_sc_scatter_deg = _sc_scatter(True)
_sc_scatter_nodeg = _sc_scatter(False)


def _mm2_body(x_ref, w0_ref, b0_ref, w1_ref, b1_ref, o0_ref, o1_ref):
    x = x_ref[...]
    o0_ref[...] = jnp.dot(x, w0_ref[...], preferred_element_type=jnp.float32) + b0_ref[...]
    o1_ref[...] = jnp.dot(x, w1_ref[...], preferred_element_type=jnp.float32) + b1_ref[...]


def _mm2(x, w0, b0, w1, b1):
    blk = pl.BlockSpec((BR, D), lambda i: (i, 0))
    wspec = pl.BlockSpec((D, D), lambda i: (0, 0))
    bspec = pl.BlockSpec((1, D), lambda i: (0, 0))
    return pl.pallas_call(
        _mm2_body,
        grid=(NB,),
        in_specs=[blk, wspec, bspec, wspec, bspec],
        out_specs=[blk, blk],
        out_shape=[jax.ShapeDtypeStruct((N, D), jnp.float32)] * 2,
    )(x, w0, b0.reshape(1, D), w1, b1.reshape(1, D))


def _fuse_factory(last):
    def body(vw0_ref, p_ref, degp_ref, g_ref, be_ref, res_ref,
             w0_ref, b0_ref, w1_ref, b1_ref, *orefs):
        if last:
            o0_ref, t_s, sums_s = orefs
            o1_ref = None
        else:
            o0_ref, o1_ref, t_s, sums_s = orefs
        p = pl.program_id(0)
        i = pl.program_id(1)

        @pl.when(p == 0)
        def _():
            deg = degp_ref[0, :, 0] + degp_ref[1, :, 0]
            dinv = 1.0 / (1.0 + deg)
            t = (vw0_ref[...] + p_ref[0] + p_ref[1]) * dinv[:, None]
            t_s[pl.ds(i * BR, BR), :] = t
            s = jnp.sum(t, axis=0)
            s2 = jnp.sum(t * t, axis=0)
            upd = jnp.concatenate(
                [s[None, :], s2[None, :], jnp.zeros((6, D), jnp.float32)], axis=0)

            @pl.when(i == 0)
            def _():
                sums_s[...] = upd

            @pl.when(i > 0)
            def _():
                sums_s[...] = sums_s[...] + upd

        @pl.when(p == 1)
        def _():
            m = sums_s[0, :] / N
            v = sums_s[1, :] / N - m * m
            scale = g_ref[0, :] * lax.rsqrt(v + EPS)
            t = t_s[pl.ds(i * BR, BR), :]
            y = (t - m[None, :]) * scale[None, :] + be_ref[0, :][None, :]
            if last:
                y = y + res_ref[...]
            y = jnp.maximum(y, 0.0)
            if last:
                o0_ref[...] = y
            else:
                o0_ref[...] = jnp.dot(
                    y, w0_ref[...], preferred_element_type=jnp.float32) + b0_ref[...]
                o1_ref[...] = jnp.dot(
                    y, w1_ref[...], preferred_element_type=jnp.float32) + b1_ref[...]

    blk = pl.BlockSpec((BR, D), lambda p, i: (i, 0))
    row = pl.BlockSpec((1, D), lambda p, i: (0, 0))
    wsp = pl.BlockSpec((D, D), lambda p, i: (0, 0))
    in_specs = [
        blk,
        pl.BlockSpec((2, BR, D), lambda p, i: (0, i, 0)),
        pl.BlockSpec((2, BR, 16), lambda p, i: (0, i, 0)),
        row, row,
        blk,
        wsp, row, wsp, row,
    ]
    n_out = 1 if last else 2
    return pl.pallas_call(
        body,
        grid=(2, NB),
        in_specs=in_specs,
        out_specs=[blk] * n_out,
        out_shape=[jax.ShapeDtypeStruct((N, D), jnp.float32)] * n_out,
        scratch_shapes=[pltpu.VMEM((N, D), jnp.float32),
                        pltpu.VMEM((8, D), jnp.float32)],
    )


_fuse_mid = _fuse_factory(False)
_fuse_last = _fuse_factory(True)


def kernel(features, edges, w0_0, b0_0, w1_0, b1_0, g_0, be_0,
           w0_1, b0_1, w1_1, b1_1, g_1, be_1,
           w0_2, b0_2, w1_2, b1_2, g_2, be_2):
    npad = ER * D - 2 * E
    srcs = jnp.concatenate(
        [edges[:, 1], edges[:, 0], jnp.zeros((npad,), jnp.int32)])
    dsts = jnp.concatenate(
        [edges[:, 0], edges[:, 1], jnp.full((npad,), N, jnp.int32)])
    srcs = srcs.reshape(ER, D)
    dsts = dsts.reshape(ER, D)
    zeros = jnp.zeros((RPS, D), jnp.float32)
    zeros16 = jnp.zeros((RPS, 16), jnp.float32)
    ones16 = jnp.ones((D, 16), jnp.float32)

    params = [(w0_1, b0_1, w1_1, b1_1, g_0, be_0),
              (w0_2, b0_2, w1_2, b1_2, g_1, be_1),
              (w0_2, b0_2, w1_2, b1_2, g_2, be_2)]

    vw0, vw1 = _mm2(features, w0_0, b0_0, w1_0, b1_0)
    degpart = None
    for li, (w0n, b0n, w1n, b1n, g, be) in enumerate(params):
        if li == 0:
            part, degpart = _sc_scatter_deg(
                vw1, srcs, dsts, zeros, zeros16, ones16)
            degpart = degpart.reshape(2, NPAD, 16)
        else:
            (part,) = _sc_scatter_nodeg(
                vw1, srcs, dsts, zeros, zeros16, ones16)
        fuse = _fuse_last if li == 2 else _fuse_mid
        outs = fuse(vw0, part.reshape(2, NPAD, D), degpart,
                    g.reshape(1, D), be.reshape(1, D), features,
                    w0n, b0n.reshape(1, D), w1n, b1n.reshape(1, D))
        if li == 2:
            (y,) = outs
            return y
        vw0, vw1 = outs


# speedup vs baseline: 1.2041x; 1.0287x over previous
import jax
import jax.numpy as jnp
from jax import lax
from jax.experimental import pallas as pl
from jax.experimental.pallas import tpu as pltpu
from jax.experimental.pallas import tpu_sc as plsc

N = 10000
E = 320000
D = 128
EPS = 1e-5

NB = 10
BR = N // NB
NW = 32
ER = 5120
RPW = ER // NW
NPAD = 10016
RPS = NPAD // 16

_mesh = plsc.VectorSubcoreMesh(core_axis_name="c", subcore_axis_name="s")


def _sc_scatter(with_deg):
    out_type = [jax.ShapeDtypeStruct((NW, RPS, D), jnp.float32)]
    scratch = [
        pltpu.VMEM_SHARED((NPAD, D), jnp.float32),
        pltpu.VMEM((D,), jnp.int32),
        pltpu.VMEM((D,), jnp.int32),
        pltpu.VMEM((D,), jnp.int32),
        pltpu.VMEM((D,), jnp.int32),
        pltpu.VMEM((D, D), jnp.float32),
        pltpu.SemaphoreType.DMA,
        pltpu.SemaphoreType.DMA,
    ]
    if with_deg:
        out_type.append(jax.ShapeDtypeStruct((NW, RPS, 16), jnp.float32))
        scratch += [
            pltpu.VMEM_SHARED((NPAD, 16), jnp.float32),
            pltpu.VMEM((D, 16), jnp.float32),
        ]

    def body(vw1, srcs, dsts, zeros, zeros16, ones_in, part, *rest):
        if with_deg:
            (degpart, acc_sh, sx0, sx1, dx0, dx1, rows, gsem, isem,
             deg_sh, ones_v) = rest
        else:
            acc_sh, sx0, sx1, dx0, dx1, rows, gsem, isem = rest
        srcx = [sx0, sx1]
        dstx = [dx0, dx1]
        cid = lax.axis_index("c")
        sid = lax.axis_index("s")
        w = cid * 16 + sid
        pltpu.sync_copy(zeros, acc_sh.at[pl.ds(sid * RPS, RPS)])
        if with_deg:
            pltpu.sync_copy(zeros16, deg_sh.at[pl.ds(sid * RPS, RPS)])
            pltpu.sync_copy(ones_in, ones_v)
        plsc.subcore_barrier()

        def idx_start(t, q):
            pltpu.async_copy(srcs.at[w + NW * t], srcx[q], isem)
            pltpu.async_copy(dsts.at[w + NW * t], dstx[q], isem)

        def idx_wait(t, q):
            pltpu.make_async_copy(srcs.at[w + NW * t], srcx[q], isem).wait()
            pltpu.make_async_copy(dsts.at[w + NW * t], dstx[q], isem).wait()

        def slot(t, q, first=False, last=False):
            if not first:
                idx_wait(t, q)
            h = pltpu.async_copy(vw1.at[srcx[q]], rows, gsem)
            if not last:
                idx_start(t + 1, 1 - q)
            h.wait()
            pltpu.sync_copy(rows, acc_sh.at[dstx[q]], add=True)
            if with_deg:
                pltpu.sync_copy(ones_v, deg_sh.at[dstx[q]], add=True)

        pltpu.sync_copy(srcs.at[w], sx0)
        pltpu.sync_copy(dsts.at[w], dx0)
        slot(0, 0, first=True)

        def two_slots(c, carry):
            slot(2 * c + 1, 1)
            slot(2 * c + 2, 0)
            return carry

        lax.fori_loop(0, RPW // 2 - 1, two_slots, 0)
        slot(RPW - 1, 1, last=True)

        plsc.subcore_barrier()
        pltpu.sync_copy(acc_sh.at[pl.ds(sid * RPS, RPS)], part.at[w])
        if with_deg:
            pltpu.sync_copy(deg_sh.at[pl.ds(sid * RPS, RPS)], degpart.at[w])

    return pl.kernel(body, out_type=out_type, mesh=_mesh, scratch_types=scratch,
                     compiler_params=pltpu.CompilerParams(use_tc_tiling_on_sc=False))


_sc_scatter_deg = _sc_scatter(True)
_sc_scatter_nodeg = _sc_scatter(False)


def _mm2_body(x_ref, w0_ref, b0_ref, w1_ref, b1_ref, o0_ref, o1_ref):
    x = x_ref[...]
    o0_ref[...] = jnp.dot(x, w0_ref[...], preferred_element_type=jnp.float32) + b0_ref[...]
    o1_ref[...] = jnp.dot(x, w1_ref[...], preferred_element_type=jnp.float32) + b1_ref[...]


def _mm2(x, w0, b0, w1, b1):
    blk = pl.BlockSpec((BR, D), lambda i: (i, 0))
    wspec = pl.BlockSpec((D, D), lambda i: (0, 0))
    bspec = pl.BlockSpec((1, D), lambda i: (0, 0))
    return pl.pallas_call(
        _mm2_body,
        grid=(NB,),
        in_specs=[blk, wspec, bspec, wspec, bspec],
        out_specs=[blk, blk],
        out_shape=[jax.ShapeDtypeStruct((N, D), jnp.float32)] * 2,
    )(x, w0, b0.reshape(1, D), w1, b1.reshape(1, D))


def _fuse_factory(last):
    def body(vw0_ref, p_ref, degp_ref, g_ref, be_ref, res_ref,
             w0_ref, b0_ref, w1_ref, b1_ref, *orefs):
        if last:
            o0_ref, t_s, sums_s = orefs
            o1_ref = None
        else:
            o0_ref, o1_ref, t_s, sums_s = orefs
        p = pl.program_id(0)
        i = pl.program_id(1)

        @pl.when(p == 0)
        def _():
            deg = degp_ref[0, :, 0] + degp_ref[1, :, 0]
            dinv = 1.0 / (1.0 + deg)
            t = (vw0_ref[...] + p_ref[0] + p_ref[1]) * dinv[:, None]
            t_s[pl.ds(i * BR, BR), :] = t
            s = jnp.sum(t, axis=0)
            s2 = jnp.sum(t * t, axis=0)
            upd = jnp.concatenate(
                [s[None, :], s2[None, :], jnp.zeros((6, D), jnp.float32)], axis=0)

            @pl.when(i == 0)
            def _():
                sums_s[...] = upd

            @pl.when(i > 0)
            def _():
                sums_s[...] = sums_s[...] + upd

        @pl.when(p == 1)
        def _():
            m = sums_s[0, :] / N
            v = sums_s[1, :] / N - m * m
            scale = g_ref[0, :] * lax.rsqrt(v + EPS)
            t = t_s[pl.ds(i * BR, BR), :]
            y = (t - m[None, :]) * scale[None, :] + be_ref[0, :][None, :]
            if last:
                y = y + res_ref[...]
            y = jnp.maximum(y, 0.0)
            if last:
                o0_ref[...] = y
            else:
                o0_ref[...] = jnp.dot(
                    y, w0_ref[...], preferred_element_type=jnp.float32) + b0_ref[...]
                o1_ref[...] = jnp.dot(
                    y, w1_ref[...], preferred_element_type=jnp.float32) + b1_ref[...]

    blk = pl.BlockSpec((BR, D), lambda p, i: (i, 0))
    row = pl.BlockSpec((1, D), lambda p, i: (0, 0))
    wsp = pl.BlockSpec((D, D), lambda p, i: (0, 0))
    in_specs = [
        blk,
        pl.BlockSpec((2, BR, D), lambda p, i: (0, i, 0)),
        pl.BlockSpec((2, BR, 16), lambda p, i: (0, i, 0)),
        row, row,
        blk,
        wsp, row, wsp, row,
    ]
    n_out = 1 if last else 2
    return pl.pallas_call(
        body,
        grid=(2, NB),
        in_specs=in_specs,
        out_specs=[blk] * n_out,
        out_shape=[jax.ShapeDtypeStruct((N, D), jnp.float32)] * n_out,
        scratch_shapes=[pltpu.VMEM((N, D), jnp.float32),
                        pltpu.VMEM((8, D), jnp.float32)],
    )


_fuse_mid = _fuse_factory(False)
_fuse_last = _fuse_factory(True)


def kernel(features, edges, w0_0, b0_0, w1_0, b1_0, g_0, be_0,
           w0_1, b0_1, w1_1, b1_1, g_1, be_1,
           w0_2, b0_2, w1_2, b1_2, g_2, be_2):
    npad = ER * D - 2 * E
    srcs = jnp.concatenate(
        [edges[:, 1], edges[:, 0], jnp.zeros((npad,), jnp.int32)])
    dsts = jnp.concatenate(
        [edges[:, 0], edges[:, 1],
         N + (jnp.arange(npad, dtype=jnp.int32) % (NPAD - N))])
    srcs = srcs.reshape(ER, D)
    dsts = dsts.reshape(ER, D)
    zeros = jnp.zeros((RPS, D), jnp.float32)
    zeros16 = jnp.zeros((RPS, 16), jnp.float32)
    ones16 = jnp.ones((D, 16), jnp.float32)

    params = [(w0_1, b0_1, w1_1, b1_1, g_0, be_0),
              (w0_2, b0_2, w1_2, b1_2, g_1, be_1),
              (w0_2, b0_2, w1_2, b1_2, g_2, be_2)]

    vw0, vw1 = _mm2(features, w0_0, b0_0, w1_0, b1_0)
    degpart = None
    for li, (w0n, b0n, w1n, b1n, g, be) in enumerate(params):
        if li == 0:
            part, degpart = _sc_scatter_deg(
                vw1, srcs, dsts, zeros, zeros16, ones16)
            degpart = degpart.reshape(2, NPAD, 16)
        else:
            (part,) = _sc_scatter_nodeg(
                vw1, srcs, dsts, zeros, zeros16, ones16)
        fuse = _fuse_last if li == 2 else _fuse_mid
        outs = fuse(vw0, part.reshape(2, NPAD, D), degpart,
                    g.reshape(1, D), be.reshape(1, D), features,
                    w0n, b0n.reshape(1, D), w1n, b1n.reshape(1, D))
        if li == 2:
            (y,) = outs
            return y
        vw0, vw1 = outs


# speedup vs baseline: 2.9852x; 2.4792x over previous
import jax
import jax.numpy as jnp
from jax import lax
from jax.experimental import pallas as pl
from jax.experimental.pallas import tpu as pltpu
from jax.experimental.pallas import tpu_sc as plsc

N = 10000
E = 320000
D = 128
EPS = 1e-5

NB = 10
BR = N // NB
NW = 32
ER = (2 * E) // D
NPAD = 10016
RPS = NPAD // 16

_mesh = plsc.VectorSubcoreMesh(core_axis_name="c", subcore_axis_name="s")


def _sc_scatter(with_deg):
    out_type = [jax.ShapeDtypeStruct((NW, RPS, D), jnp.float32)]
    scratch = [
        pltpu.VMEM_SHARED((NPAD, D), jnp.float32),
        pltpu.VMEM((D,), jnp.int32),
        pltpu.VMEM((D,), jnp.int32),
        pltpu.VMEM((D,), jnp.int32),
        pltpu.VMEM((D,), jnp.int32),
        pltpu.VMEM((D, D), jnp.float32),
        pltpu.SemaphoreType.DMA,
        pltpu.SemaphoreType.DMA,
    ]
    if with_deg:
        out_type.append(jax.ShapeDtypeStruct((NW, RPS, 16), jnp.float32))
        scratch += [
            pltpu.VMEM_SHARED((NPAD, 16), jnp.float32),
            pltpu.VMEM((D, 16), jnp.float32),
        ]

    def body(vw1, srcs, dsts, zeros, zeros16, ones_in, part, *rest):
        if with_deg:
            (degpart, acc_sh, sx0, sx1, dx0, dx1, rows, gsem, isem,
             deg_sh, ones_v) = rest
        else:
            acc_sh, sx0, sx1, dx0, dx1, rows, gsem, isem = rest
        srcx = [sx0, sx1]
        dstx = [dx0, dx1]
        cid = lax.axis_index("c")
        sid = lax.axis_index("s")
        w = cid * 16 + sid
        pltpu.sync_copy(zeros, acc_sh.at[pl.ds(sid * RPS, RPS)])
        if with_deg:
            pltpu.sync_copy(zeros16, deg_sh.at[pl.ds(sid * RPS, RPS)])
            pltpu.sync_copy(ones_in, ones_v)
        plsc.subcore_barrier()

        def idx_start(t, q):
            r = jnp.minimum(w + NW * t, ER - 1)
            pltpu.async_copy(srcs.at[r], srcx[q], isem)
            pltpu.async_copy(dsts.at[r], dstx[q], isem)

        def idx_wait(t, q):
            r = jnp.minimum(w + NW * t, ER - 1)
            pltpu.make_async_copy(srcs.at[r], srcx[q], isem).wait()
            pltpu.make_async_copy(dsts.at[r], dstx[q], isem).wait()

        def slot(t, q, first=False, last=False):
            if not first:
                idx_wait(t, q)
            h = pltpu.async_copy(vw1.at[srcx[q]], rows, gsem)
            if not last:
                idx_start(t + 1, 1 - q)
            h.wait()
            pltpu.sync_copy(rows, acc_sh.at[dstx[q]], add=True)
            if with_deg:
                pltpu.sync_copy(ones_v, deg_sh.at[dstx[q]], add=True)

        pltpu.sync_copy(srcs.at[w], sx0)
        pltpu.sync_copy(dsts.at[w], dx0)
        slot(0, 0, first=True)

        def two_slots(c, carry):
            slot(2 * c + 1, 1)
            slot(2 * c + 2, 0)
            return carry

        lax.fori_loop(0, 77, two_slots, 0)
        slot(155, 1)

        @pl.when(w < ER - 156 * NW)
        def _():
            slot(156, 0, last=True)

        @pl.when(w >= ER - 156 * NW)
        def _():
            idx_wait(156, 0)

        plsc.subcore_barrier()
        pltpu.sync_copy(acc_sh.at[pl.ds(sid * RPS, RPS)], part.at[w])
        if with_deg:
            pltpu.sync_copy(deg_sh.at[pl.ds(sid * RPS, RPS)], degpart.at[w])

    return pl.kernel(body, out_type=out_type, mesh=_mesh, scratch_types=scratch,
                     compiler_params=pltpu.CompilerParams(use_tc_tiling_on_sc=False))


_sc_scatter_deg = _sc_scatter(True)
_sc_scatter_nodeg = _sc_scatter(False)


def _mm2_body(x_ref, w0_ref, b0_ref, w1_ref, b1_ref, o0_ref, o1_ref):
    x = x_ref[...]
    o0_ref[...] = jnp.dot(x, w0_ref[...], preferred_element_type=jnp.float32) + b0_ref[...]
    o1_ref[...] = jnp.dot(x, w1_ref[...], preferred_element_type=jnp.float32) + b1_ref[...]


def _mm2(x, w0, b0, w1, b1):
    blk = pl.BlockSpec((BR, D), lambda i: (i, 0))
    wspec = pl.BlockSpec((D, D), lambda i: (0, 0))
    bspec = pl.BlockSpec((1, D), lambda i: (0, 0))
    return pl.pallas_call(
        _mm2_body,
        grid=(NB,),
        in_specs=[blk, wspec, bspec, wspec, bspec],
        out_specs=[blk, blk],
        out_shape=[jax.ShapeDtypeStruct((N, D), jnp.float32)] * 2,
    )(x, w0, b0.reshape(1, D), w1, b1.reshape(1, D))


def _fuse_factory(last):
    def body(vw0_ref, p_ref, degp_ref, g_ref, be_ref, res_ref,
             w0_ref, b0_ref, w1_ref, b1_ref, *orefs):
        if last:
            o0_ref, t_s, sums_s = orefs
            o1_ref = None
        else:
            o0_ref, o1_ref, t_s, sums_s = orefs
        p = pl.program_id(0)
        i = pl.program_id(1)

        @pl.when(p == 0)
        def _():
            deg = degp_ref[0, :, 0] + degp_ref[1, :, 0]
            dinv = 1.0 / (1.0 + deg)
            t = (vw0_ref[...] + p_ref[0] + p_ref[1]) * dinv[:, None]
            t_s[pl.ds(i * BR, BR), :] = t
            s = jnp.sum(t, axis=0)
            s2 = jnp.sum(t * t, axis=0)
            upd = jnp.concatenate(
                [s[None, :], s2[None, :], jnp.zeros((6, D), jnp.float32)], axis=0)

            @pl.when(i == 0)
            def _():
                sums_s[...] = upd

            @pl.when(i > 0)
            def _():
                sums_s[...] = sums_s[...] + upd

        @pl.when(p == 1)
        def _():
            m = sums_s[0, :] / N
            v = sums_s[1, :] / N - m * m
            scale = g_ref[0, :] * lax.rsqrt(v + EPS)
            t = t_s[pl.ds(i * BR, BR), :]
            y = (t - m[None, :]) * scale[None, :] + be_ref[0, :][None, :]
            if last:
                y = y + res_ref[...]
            y = jnp.maximum(y, 0.0)
            if last:
                o0_ref[...] = y
            else:
                o0_ref[...] = jnp.dot(
                    y, w0_ref[...], preferred_element_type=jnp.float32) + b0_ref[...]
                o1_ref[...] = jnp.dot(
                    y, w1_ref[...], preferred_element_type=jnp.float32) + b1_ref[...]

    blk = pl.BlockSpec((BR, D), lambda p, i: (i, 0))
    row = pl.BlockSpec((1, D), lambda p, i: (0, 0))
    wsp = pl.BlockSpec((D, D), lambda p, i: (0, 0))
    in_specs = [
        blk,
        pl.BlockSpec((2, BR, D), lambda p, i: (0, i, 0)),
        pl.BlockSpec((2, BR, 16), lambda p, i: (0, i, 0)),
        row, row,
        blk,
        wsp, row, wsp, row,
    ]
    n_out = 1 if last else 2
    return pl.pallas_call(
        body,
        grid=(2, NB),
        in_specs=in_specs,
        out_specs=[blk] * n_out,
        out_shape=[jax.ShapeDtypeStruct((N, D), jnp.float32)] * n_out,
        scratch_shapes=[pltpu.VMEM((N, D), jnp.float32),
                        pltpu.VMEM((8, D), jnp.float32)],
    )


_fuse_mid = _fuse_factory(False)
_fuse_last = _fuse_factory(True)


def kernel(features, edges, w0_0, b0_0, w1_0, b1_0, g_0, be_0,
           w0_1, b0_1, w1_1, b1_1, g_1, be_1,
           w0_2, b0_2, w1_2, b1_2, g_2, be_2):
    srcs = jnp.concatenate([edges[:, 1], edges[:, 0]]).reshape(ER, D)
    dsts = jnp.concatenate([edges[:, 0], edges[:, 1]]).reshape(ER, D)
    zeros = jnp.zeros((RPS, D), jnp.float32)
    zeros16 = jnp.zeros((RPS, 16), jnp.float32)
    ones16 = jnp.ones((D, 16), jnp.float32)

    params = [(w0_1, b0_1, w1_1, b1_1, g_0, be_0),
              (w0_2, b0_2, w1_2, b1_2, g_1, be_1),
              (w0_2, b0_2, w1_2, b1_2, g_2, be_2)]

    vw0, vw1 = _mm2(features, w0_0, b0_0, w1_0, b1_0)
    degpart = None
    for li, (w0n, b0n, w1n, b1n, g, be) in enumerate(params):
        if li == 0:
            part, degpart = _sc_scatter_deg(
                vw1, srcs, dsts, zeros, zeros16, ones16)
            degpart = degpart.reshape(2, NPAD, 16)
        else:
            (part,) = _sc_scatter_nodeg(
                vw1, srcs, dsts, zeros, zeros16, ones16)
        fuse = _fuse_last if li == 2 else _fuse_mid
        outs = fuse(vw0, part.reshape(2, NPAD, D), degpart,
                    g.reshape(1, D), be.reshape(1, D), features,
                    w0n, b0n.reshape(1, D), w1n, b1n.reshape(1, D))
        if li == 2:
            (y,) = outs
            return y
        vw0, vw1 = outs


# speedup vs baseline: 3.8746x; 1.2979x over previous
import jax
import jax.numpy as jnp
from jax import lax
from jax.experimental import pallas as pl
from jax.experimental.pallas import tpu as pltpu
from jax.experimental.pallas import tpu_sc as plsc

N = 10000
E = 320000
D = 128
EPS = 1e-5

NB = 10
BR = N // NB
NW = 32
ER = (2 * E) // D
NPAD = 10016
RPS = NPAD // 16

_mesh = plsc.VectorSubcoreMesh(core_axis_name="c", subcore_axis_name="s")


def _sc_scatter(with_deg):
    out_type = [jax.ShapeDtypeStruct((NW, RPS, D), jnp.float32)]
    scratch = [
        pltpu.VMEM_SHARED((NPAD, D), jnp.float32),
        pltpu.VMEM((D,), jnp.int32),
        pltpu.VMEM((D,), jnp.int32),
        pltpu.VMEM((D,), jnp.int32),
        pltpu.VMEM((D,), jnp.int32),
        pltpu.VMEM((D, D), jnp.float32),
        pltpu.VMEM((D, D), jnp.float32),
        pltpu.SemaphoreType.DMA,
        pltpu.SemaphoreType.DMA,
        pltpu.SemaphoreType.DMA,
        pltpu.SemaphoreType.DMA,
        pltpu.SemaphoreType.DMA,
        pltpu.SemaphoreType.DMA,
        pltpu.SemaphoreType.DMA,
        pltpu.SemaphoreType.DMA,
    ]
    if with_deg:
        out_type.append(jax.ShapeDtypeStruct((NW, RPS, 16), jnp.float32))
        scratch += [
            pltpu.VMEM_SHARED((NPAD, 16), jnp.float32),
            pltpu.VMEM((D, 16), jnp.float32),
        ]

    def body(vw1, srcs, dsts, zeros, zeros16, ones_in, part, *rest):
        if with_deg:
            (degpart, acc_sh, sx0, sx1, dx0, dx1, r0, r1,
             g0, g1, s0, s1, is0, is1, id0, id1, deg_sh, ones_v) = rest
        else:
            (acc_sh, sx0, sx1, dx0, dx1, r0, r1,
             g0, g1, s0, s1, is0, is1, id0, id1) = rest
        srcx = [sx0, sx1]
        dstx = [dx0, dx1]
        rows = [r0, r1]
        gsem = [g0, g1]
        ssem = [s0, s1]
        isems = [is0, is1]
        isemd = [id0, id1]
        cid = lax.axis_index("c")
        sid = lax.axis_index("s")
        w = cid * 16 + sid
        pltpu.sync_copy(zeros, acc_sh.at[pl.ds(sid * RPS, RPS)])
        if with_deg:
            pltpu.sync_copy(zeros16, deg_sh.at[pl.ds(sid * RPS, RPS)])
            pltpu.sync_copy(ones_in, ones_v)
        plsc.subcore_barrier()

        def _row(t):
            return jnp.minimum(w + NW * t, ER - 1)

        def srcx_start(t, q):
            pltpu.async_copy(srcs.at[_row(t)], srcx[q], isems[q])

        def srcx_wait(t, q):
            pltpu.make_async_copy(srcs.at[_row(t)], srcx[q], isems[q]).wait()

        def dstx_start(t, q):
            pltpu.async_copy(dsts.at[_row(t)], dstx[q], isemd[q])

        def dstx_wait(t, q):
            pltpu.make_async_copy(dsts.at[_row(t)], dstx[q], isemd[q]).wait()

        def gather_start(q):
            pltpu.async_copy(vw1.at[srcx[q]], rows[q], gsem[q])

        def gather_wait(q):
            pltpu.make_async_copy(vw1.at[srcx[q]], rows[q], gsem[q]).wait()

        def scatter_start(q):
            pltpu.async_copy(rows[q], acc_sh.at[dstx[q]], ssem[q], add=True)

        def scatter_wait(q):
            pltpu.make_async_copy(rows[q], acc_sh.at[dstx[q]], ssem[q]).wait()

        def slot(t, q, first=False):
            gather_wait(q)
            if not first:
                dstx_wait(t, q)
            scatter_start(q)
            if with_deg:
                pltpu.sync_copy(ones_v, deg_sh.at[dstx[q]], add=True)
            srcx_start(t + 2, q)
            if not first:
                scatter_wait(1 - q)
            dstx_start(t + 1, 1 - q)
            srcx_wait(t + 1, 1 - q)
            gather_start(1 - q)

        pltpu.sync_copy(srcs.at[w], sx0)
        pltpu.sync_copy(dsts.at[w], dx0)
        gather_start(0)
        srcx_start(1, 1)
        slot(0, 0, first=True)

        def two_slots(c, carry):
            slot(2 * c + 1, 1)
            slot(2 * c + 2, 0)
            return carry

        lax.fori_loop(0, 77, two_slots, 0)
        slot(155, 1)

        @pl.when(w < ER - 156 * NW)
        def _():
            gather_wait(0)
            dstx_wait(156, 0)
            scatter_start(0)
            if with_deg:
                pltpu.sync_copy(ones_v, deg_sh.at[dstx[0]], add=True)
            scatter_wait(1)
            srcx_wait(157, 1)
            scatter_wait(0)

        @pl.when(w >= ER - 156 * NW)
        def _():
            gather_wait(0)
            dstx_wait(156, 0)
            scatter_wait(1)
            srcx_wait(157, 1)

        plsc.subcore_barrier()
        pltpu.sync_copy(acc_sh.at[pl.ds(sid * RPS, RPS)], part.at[w])
        if with_deg:
            pltpu.sync_copy(deg_sh.at[pl.ds(sid * RPS, RPS)], degpart.at[w])

    return pl.kernel(body, out_type=out_type, mesh=_mesh, scratch_types=scratch,
                     compiler_params=pltpu.CompilerParams(use_tc_tiling_on_sc=False))


_sc_scatter_deg = _sc_scatter(True)
_sc_scatter_nodeg = _sc_scatter(False)


def _mm2_body(x_ref, w0_ref, b0_ref, w1_ref, b1_ref, o0_ref, o1_ref):
    x = x_ref[...]
    o0_ref[...] = jnp.dot(x, w0_ref[...], preferred_element_type=jnp.float32) + b0_ref[...]
    o1_ref[...] = jnp.dot(x, w1_ref[...], preferred_element_type=jnp.float32) + b1_ref[...]


def _mm2(x, w0, b0, w1, b1):
    blk = pl.BlockSpec((BR, D), lambda i: (i, 0))
    wspec = pl.BlockSpec((D, D), lambda i: (0, 0))
    bspec = pl.BlockSpec((1, D), lambda i: (0, 0))
    return pl.pallas_call(
        _mm2_body,
        grid=(NB,),
        in_specs=[blk, wspec, bspec, wspec, bspec],
        out_specs=[blk, blk],
        out_shape=[jax.ShapeDtypeStruct((N, D), jnp.float32)] * 2,
    )(x, w0, b0.reshape(1, D), w1, b1.reshape(1, D))


def _fuse_factory(last):
    def body(vw0_ref, p_ref, degp_ref, g_ref, be_ref, res_ref,
             w0_ref, b0_ref, w1_ref, b1_ref, *orefs):
        if last:
            o0_ref, t_s, sums_s = orefs
            o1_ref = None
        else:
            o0_ref, o1_ref, t_s, sums_s = orefs
        p = pl.program_id(0)
        i = pl.program_id(1)

        @pl.when(p == 0)
        def _():
            deg = degp_ref[0, :, 0] + degp_ref[1, :, 0]
            dinv = 1.0 / (1.0 + deg)
            t = (vw0_ref[...] + p_ref[0] + p_ref[1]) * dinv[:, None]
            t_s[pl.ds(i * BR, BR), :] = t
            s = jnp.sum(t, axis=0)
            s2 = jnp.sum(t * t, axis=0)
            upd = jnp.concatenate(
                [s[None, :], s2[None, :], jnp.zeros((6, D), jnp.float32)], axis=0)

            @pl.when(i == 0)
            def _():
                sums_s[...] = upd

            @pl.when(i > 0)
            def _():
                sums_s[...] = sums_s[...] + upd

        @pl.when(p == 1)
        def _():
            m = sums_s[0, :] / N
            v = sums_s[1, :] / N - m * m
            scale = g_ref[0, :] * lax.rsqrt(v + EPS)
            t = t_s[pl.ds(i * BR, BR), :]
            y = (t - m[None, :]) * scale[None, :] + be_ref[0, :][None, :]
            if last:
                y = y + res_ref[...]
            y = jnp.maximum(y, 0.0)
            if last:
                o0_ref[...] = y
            else:
                o0_ref[...] = jnp.dot(
                    y, w0_ref[...], preferred_element_type=jnp.float32) + b0_ref[...]
                o1_ref[...] = jnp.dot(
                    y, w1_ref[...], preferred_element_type=jnp.float32) + b1_ref[...]

    blk = pl.BlockSpec((BR, D), lambda p, i: (i, 0))
    row = pl.BlockSpec((1, D), lambda p, i: (0, 0))
    wsp = pl.BlockSpec((D, D), lambda p, i: (0, 0))
    in_specs = [
        blk,
        pl.BlockSpec((2, BR, D), lambda p, i: (0, i, 0)),
        pl.BlockSpec((2, BR, 16), lambda p, i: (0, i, 0)),
        row, row,
        blk,
        wsp, row, wsp, row,
    ]
    n_out = 1 if last else 2
    return pl.pallas_call(
        body,
        grid=(2, NB),
        in_specs=in_specs,
        out_specs=[blk] * n_out,
        out_shape=[jax.ShapeDtypeStruct((N, D), jnp.float32)] * n_out,
        scratch_shapes=[pltpu.VMEM((N, D), jnp.float32),
                        pltpu.VMEM((8, D), jnp.float32)],
    )


_fuse_mid = _fuse_factory(False)
_fuse_last = _fuse_factory(True)


def kernel(features, edges, w0_0, b0_0, w1_0, b1_0, g_0, be_0,
           w0_1, b0_1, w1_1, b1_1, g_1, be_1,
           w0_2, b0_2, w1_2, b1_2, g_2, be_2):
    srcs = jnp.concatenate([edges[:, 1], edges[:, 0]]).reshape(ER, D)
    dsts = jnp.concatenate([edges[:, 0], edges[:, 1]]).reshape(ER, D)
    zeros = jnp.zeros((RPS, D), jnp.float32)
    zeros16 = jnp.zeros((RPS, 16), jnp.float32)
    ones16 = jnp.ones((D, 16), jnp.float32)

    params = [(w0_1, b0_1, w1_1, b1_1, g_0, be_0),
              (w0_2, b0_2, w1_2, b1_2, g_1, be_1),
              (w0_2, b0_2, w1_2, b1_2, g_2, be_2)]

    vw0, vw1 = _mm2(features, w0_0, b0_0, w1_0, b1_0)
    degpart = None
    for li, (w0n, b0n, w1n, b1n, g, be) in enumerate(params):
        if li == 0:
            part, degpart = _sc_scatter_deg(
                vw1, srcs, dsts, zeros, zeros16, ones16)
            degpart = degpart.reshape(2, NPAD, 16)
        else:
            (part,) = _sc_scatter_nodeg(
                vw1, srcs, dsts, zeros, zeros16, ones16)
        fuse = _fuse_last if li == 2 else _fuse_mid
        outs = fuse(vw0, part.reshape(2, NPAD, D), degpart,
                    g.reshape(1, D), be.reshape(1, D), features,
                    w0n, b0n.reshape(1, D), w1n, b1n.reshape(1, D))
        if li == 2:
            (y,) = outs
            return y
        vw0, vw1 = outs


# speedup vs baseline: 3.9907x; 1.0300x over previous
import jax
import jax.numpy as jnp
from jax import lax
from jax.experimental import pallas as pl
from jax.experimental.pallas import tpu as pltpu
from jax.experimental.pallas import tpu_sc as plsc

N = 10000
E = 320000
D = 128
EPS = 1e-5

NB = 10
BR = N // NB
NW = 32
ER = (2 * E) // D
NPAD = 10016
RPS = NPAD // 16

_mesh = plsc.VectorSubcoreMesh(core_axis_name="c", subcore_axis_name="s")


def _sc_scatter(with_deg):
    out_type = [jax.ShapeDtypeStruct((NW, RPS, D), jnp.float32)]
    scratch = [
        pltpu.VMEM_SHARED((NPAD, D), jnp.float32),
        pltpu.VMEM((D,), jnp.int32),
        pltpu.VMEM((D,), jnp.int32),
        pltpu.VMEM((D,), jnp.int32),
        pltpu.VMEM((D,), jnp.int32),
        pltpu.VMEM((D, D), jnp.float32),
        pltpu.VMEM((D, D), jnp.float32),
        pltpu.SemaphoreType.DMA,
        pltpu.SemaphoreType.DMA,
        pltpu.SemaphoreType.DMA,
        pltpu.SemaphoreType.DMA,
        pltpu.SemaphoreType.DMA,
        pltpu.SemaphoreType.DMA,
        pltpu.SemaphoreType.DMA,
        pltpu.SemaphoreType.DMA,
    ]
    if with_deg:
        out_type.append(jax.ShapeDtypeStruct((NW, RPS, 16), jnp.float32))
        scratch += [
            pltpu.VMEM_SHARED((NPAD, 16), jnp.float32),
            pltpu.VMEM((D, 16), jnp.float32),
            pltpu.SemaphoreType.DMA,
            pltpu.SemaphoreType.DMA,
        ]

    def body(vw1, srcs, dsts, zeros, zeros16, ones_in, part, *rest):
        if with_deg:
            (degpart, acc_sh, sx0, sx1, dx0, dx1, r0, r1,
             g0, g1, s0, s1, is0, is1, id0, id1, deg_sh, ones_v,
             dg0, dg1) = rest
            dsem = [dg0, dg1]
        else:
            (acc_sh, sx0, sx1, dx0, dx1, r0, r1,
             g0, g1, s0, s1, is0, is1, id0, id1) = rest
        srcx = [sx0, sx1]
        dstx = [dx0, dx1]
        rows = [r0, r1]
        gsem = [g0, g1]
        ssem = [s0, s1]
        isems = [is0, is1]
        isemd = [id0, id1]
        cid = lax.axis_index("c")
        sid = lax.axis_index("s")
        w = cid * 16 + sid
        pltpu.sync_copy(zeros, acc_sh.at[pl.ds(sid * RPS, RPS)])
        if with_deg:
            pltpu.sync_copy(zeros16, deg_sh.at[pl.ds(sid * RPS, RPS)])
            pltpu.sync_copy(ones_in, ones_v)
        plsc.subcore_barrier()

        def _row(t):
            return jnp.minimum(w + NW * t, ER - 1)

        def srcx_start(t, q):
            pltpu.async_copy(srcs.at[_row(t)], srcx[q], isems[q])

        def srcx_wait(t, q):
            pltpu.make_async_copy(srcs.at[_row(t)], srcx[q], isems[q]).wait()

        def dstx_start(t, q):
            pltpu.async_copy(dsts.at[_row(t)], dstx[q], isemd[q])

        def dstx_wait(t, q):
            pltpu.make_async_copy(dsts.at[_row(t)], dstx[q], isemd[q]).wait()

        def gather_start(q):
            pltpu.async_copy(vw1.at[srcx[q]], rows[q], gsem[q])

        def gather_wait(q):
            pltpu.make_async_copy(vw1.at[srcx[q]], rows[q], gsem[q]).wait()

        def scatter_start(q):
            pltpu.async_copy(rows[q], acc_sh.at[dstx[q]], ssem[q], add=True)

        def scatter_wait(q):
            pltpu.make_async_copy(rows[q], acc_sh.at[dstx[q]], ssem[q]).wait()

        def deg_start(q):
            pltpu.async_copy(ones_v, deg_sh.at[dstx[q]], dsem[q], add=True)

        def deg_wait(q):
            pltpu.make_async_copy(ones_v, deg_sh.at[dstx[q]], dsem[q]).wait()

        def slot(t, q, first=False):
            gather_wait(q)
            if not first:
                dstx_wait(t, q)
            scatter_start(q)
            if with_deg:
                deg_start(q)
            srcx_start(t + 2, q)
            if not first:
                scatter_wait(1 - q)
                if with_deg:
                    deg_wait(1 - q)
            dstx_start(t + 1, 1 - q)
            srcx_wait(t + 1, 1 - q)
            gather_start(1 - q)

        pltpu.sync_copy(srcs.at[w], sx0)
        pltpu.sync_copy(dsts.at[w], dx0)
        gather_start(0)
        srcx_start(1, 1)
        slot(0, 0, first=True)

        def two_slots(c, carry):
            slot(2 * c + 1, 1)
            slot(2 * c + 2, 0)
            return carry

        lax.fori_loop(0, 77, two_slots, 0)
        slot(155, 1)

        @pl.when(w < ER - 156 * NW)
        def _():
            gather_wait(0)
            dstx_wait(156, 0)
            scatter_start(0)
            if with_deg:
                deg_start(0)
            scatter_wait(1)
            srcx_wait(157, 1)
            scatter_wait(0)
            if with_deg:
                deg_wait(1)
                deg_wait(0)

        @pl.when(w >= ER - 156 * NW)
        def _():
            gather_wait(0)
            dstx_wait(156, 0)
            scatter_wait(1)
            srcx_wait(157, 1)
            if with_deg:
                deg_wait(1)

        plsc.subcore_barrier()
        pltpu.sync_copy(acc_sh.at[pl.ds(sid * RPS, RPS)], part.at[w])
        if with_deg:
            pltpu.sync_copy(deg_sh.at[pl.ds(sid * RPS, RPS)], degpart.at[w])

    return pl.kernel(body, out_type=out_type, mesh=_mesh, scratch_types=scratch,
                     compiler_params=pltpu.CompilerParams(use_tc_tiling_on_sc=False))


_sc_scatter_deg = _sc_scatter(True)
_sc_scatter_nodeg = _sc_scatter(False)


def _mm2_body(x_ref, w0_ref, b0_ref, w1_ref, b1_ref, o0_ref, o1_ref):
    x = x_ref[...]
    o0_ref[...] = jnp.dot(x, w0_ref[...], preferred_element_type=jnp.float32) + b0_ref[...]
    o1_ref[...] = jnp.dot(x, w1_ref[...], preferred_element_type=jnp.float32) + b1_ref[...]


def _mm2(x, w0, b0, w1, b1):
    blk = pl.BlockSpec((BR, D), lambda i: (i, 0))
    wspec = pl.BlockSpec((D, D), lambda i: (0, 0))
    bspec = pl.BlockSpec((1, D), lambda i: (0, 0))
    return pl.pallas_call(
        _mm2_body,
        grid=(NB,),
        in_specs=[blk, wspec, bspec, wspec, bspec],
        out_specs=[blk, blk],
        out_shape=[jax.ShapeDtypeStruct((N, D), jnp.float32)] * 2,
    )(x, w0, b0.reshape(1, D), w1, b1.reshape(1, D))


def _fuse_factory(last):
    def body(vw0_ref, p_ref, degp_ref, g_ref, be_ref, res_ref,
             w0_ref, b0_ref, w1_ref, b1_ref, *orefs):
        if last:
            o0_ref, t_s, sums_s = orefs
            o1_ref = None
        else:
            o0_ref, o1_ref, t_s, sums_s = orefs
        p = pl.program_id(0)
        i = pl.program_id(1)

        @pl.when(p == 0)
        def _():
            deg = degp_ref[0, :, 0] + degp_ref[1, :, 0]
            dinv = 1.0 / (1.0 + deg)
            t = (vw0_ref[...] + p_ref[0] + p_ref[1]) * dinv[:, None]
            t_s[pl.ds(i * BR, BR), :] = t
            s = jnp.sum(t, axis=0)
            s2 = jnp.sum(t * t, axis=0)
            upd = jnp.concatenate(
                [s[None, :], s2[None, :], jnp.zeros((6, D), jnp.float32)], axis=0)

            @pl.when(i == 0)
            def _():
                sums_s[...] = upd

            @pl.when(i > 0)
            def _():
                sums_s[...] = sums_s[...] + upd

        @pl.when(p == 1)
        def _():
            m = sums_s[0, :] / N
            v = sums_s[1, :] / N - m * m
            scale = g_ref[0, :] * lax.rsqrt(v + EPS)
            t = t_s[pl.ds(i * BR, BR), :]
            y = (t - m[None, :]) * scale[None, :] + be_ref[0, :][None, :]
            if last:
                y = y + res_ref[...]
            y = jnp.maximum(y, 0.0)
            if last:
                o0_ref[...] = y
            else:
                o0_ref[...] = jnp.dot(
                    y, w0_ref[...], preferred_element_type=jnp.float32) + b0_ref[...]
                o1_ref[...] = jnp.dot(
                    y, w1_ref[...], preferred_element_type=jnp.float32) + b1_ref[...]

    blk = pl.BlockSpec((BR, D), lambda p, i: (i, 0))
    row = pl.BlockSpec((1, D), lambda p, i: (0, 0))
    wsp = pl.BlockSpec((D, D), lambda p, i: (0, 0))
    in_specs = [
        blk,
        pl.BlockSpec((2, BR, D), lambda p, i: (0, i, 0)),
        pl.BlockSpec((2, BR, 16), lambda p, i: (0, i, 0)),
        row, row,
        blk,
        wsp, row, wsp, row,
    ]
    n_out = 1 if last else 2
    return pl.pallas_call(
        body,
        grid=(2, NB),
        in_specs=in_specs,
        out_specs=[blk] * n_out,
        out_shape=[jax.ShapeDtypeStruct((N, D), jnp.float32)] * n_out,
        scratch_shapes=[pltpu.VMEM((N, D), jnp.float32),
                        pltpu.VMEM((8, D), jnp.float32)],
    )


_fuse_mid = _fuse_factory(False)
_fuse_last = _fuse_factory(True)


def kernel(features, edges, w0_0, b0_0, w1_0, b1_0, g_0, be_0,
           w0_1, b0_1, w1_1, b1_1, g_1, be_1,
           w0_2, b0_2, w1_2, b1_2, g_2, be_2):
    srcs = jnp.concatenate([edges[:, 1], edges[:, 0]]).reshape(ER, D)
    dsts = jnp.concatenate([edges[:, 0], edges[:, 1]]).reshape(ER, D)
    zeros = jnp.zeros((RPS, D), jnp.float32)
    zeros16 = jnp.zeros((RPS, 16), jnp.float32)
    ones16 = jnp.ones((D, 16), jnp.float32)

    params = [(w0_1, b0_1, w1_1, b1_1, g_0, be_0),
              (w0_2, b0_2, w1_2, b1_2, g_1, be_1),
              (w0_2, b0_2, w1_2, b1_2, g_2, be_2)]

    vw0, vw1 = _mm2(features, w0_0, b0_0, w1_0, b1_0)
    degpart = None
    for li, (w0n, b0n, w1n, b1n, g, be) in enumerate(params):
        if li == 0:
            part, degpart = _sc_scatter_deg(
                vw1, srcs, dsts, zeros, zeros16, ones16)
            degpart = degpart.reshape(2, NPAD, 16)
        else:
            (part,) = _sc_scatter_nodeg(
                vw1, srcs, dsts, zeros, zeros16, ones16)
        fuse = _fuse_last if li == 2 else _fuse_mid
        outs = fuse(vw0, part.reshape(2, NPAD, D), degpart,
                    g.reshape(1, D), be.reshape(1, D), features,
                    w0n, b0n.reshape(1, D), w1n, b1n.reshape(1, D))
        if li == 2:
            (y,) = outs
            return y
        vw0, vw1 = outs
